# Initial kernel scaffold; baseline (speedup 1.0000x reference)
import jax
import jax.numpy as jnp
from jax.experimental import pallas as pl


def kernel(input_words, W):
    raise NotImplementedError("write your pallas kernel here")


# speedup vs baseline: 1.2045x; 1.2045x over previous
import jax
import jax.numpy as jnp
from jax.experimental import pallas as pl


def _add_kernel(x_ref, w_ref, o_ref):
    o_ref[...] = x_ref[...] + w_ref[...][None, :, :]


def kernel(input_words, W):
    batch, seq_len, emb = input_words.shape
    bb = 32
    grid = (batch // bb,)
    return pl.pallas_call(
        _add_kernel,
        grid=grid,
        in_specs=[
            pl.BlockSpec((bb, seq_len, emb), lambda i: (i, 0, 0)),
            pl.BlockSpec((seq_len, emb), lambda i: (0, 0)),
        ],
        out_specs=pl.BlockSpec((bb, seq_len, emb), lambda i: (i, 0, 0)),
        out_shape=jax.ShapeDtypeStruct((batch, seq_len, emb), input_words.dtype),
    )(input_words, W)


# speedup vs baseline: 1.2446x; 1.0333x over previous
import jax
import jax.numpy as jnp
from jax.experimental import pallas as pl


def _add_kernel(x_ref, w_ref, o_ref):
    o_ref[...] = x_ref[...] + w_ref[...][None, :, :]


def kernel(input_words, W):
    batch, seq_len, emb = input_words.shape
    bb = 64
    grid = (batch // bb,)
    return pl.pallas_call(
        _add_kernel,
        grid=grid,
        in_specs=[
            pl.BlockSpec((bb, seq_len, emb), lambda i: (i, 0, 0)),
            pl.BlockSpec((seq_len, emb), lambda i: (0, 0)),
        ],
        out_specs=pl.BlockSpec((bb, seq_len, emb), lambda i: (i, 0, 0)),
        out_shape=jax.ShapeDtypeStruct((batch, seq_len, emb), input_words.dtype),
    )(input_words, W)


# speedup vs baseline: 1.2757x; 1.0249x over previous
import jax
import jax.numpy as jnp
from jax.experimental import pallas as pl


def _add_kernel(x_ref, w_ref, o_ref):
    o_ref[...] = x_ref[...] + w_ref[...][None, :, :]


def kernel(input_words, W):
    batch, seq_len, emb = input_words.shape
    bb = 128
    grid = (batch // bb,)
    return pl.pallas_call(
        _add_kernel,
        grid=grid,
        in_specs=[
            pl.BlockSpec((bb, seq_len, emb), lambda i: (i, 0, 0)),
            pl.BlockSpec((seq_len, emb), lambda i: (0, 0)),
        ],
        out_specs=pl.BlockSpec((bb, seq_len, emb), lambda i: (i, 0, 0)),
        out_shape=jax.ShapeDtypeStruct((batch, seq_len, emb), input_words.dtype),
    )(input_words, W)
